# baseline (speedup 1.0000x reference)
import functools

import jax
import jax.numpy as jnp
from jax import lax
from jax.experimental import pallas as pl
from jax.experimental.pallas import tpu as pltpu
from jax.experimental.pallas import tpu_sc as plsc

D = 64
K = 16
PB = 256
PP = 256

_DEF = jax.lax.Precision.DEFAULT


def _dot(a, b):
    return jax.lax.dot_general(
        a, b, (((1,), (0,)), ((), ())),
        precision=_DEF, preferred_element_type=jnp.float32)


def _knn_body(xyzp_ref, xTe_ref, xTo_ref, feat_ref, fc1w_ref, fc1b_ref,
              wq_ref, d1w_ref, idx_ref, x_ref, qe_ref):
    h = xTe_ref.shape[1]
    xi = xyzp_ref[...]
    xTe = xTe_ref[...]
    xTo = xTo_ref[...]
    rsum_i = jnp.sum(xi * xi, axis=1, keepdims=True)
    d_e = rsum_i + jnp.sum(xTe * xTe, axis=0, keepdims=True) \
        - 2.0 * _dot(xi, xTe)
    d_o = rsum_i + jnp.sum(xTo * xTo, axis=0, keepdims=True) \
        - 2.0 * _dot(xi, xTo)
    iota = jax.lax.broadcasted_iota(jnp.int32, (PB, h), 1)
    le = d_e <= d_o
    P = jnp.where(le, d_e, d_o)
    lo = jnp.where(le, jnp.int32(0), jnp.int32(1))
    I = iota * 2 + lo
    L = jnp.where(le, d_o, d_e)
    LI = iota * 2 + (1 - lo)
    cols = []
    for _ in range(K):
        pos = jnp.argmin(P, axis=1)[:, None]
        eq = iota == pos
        cols.append(jnp.min(jnp.where(eq, I, jnp.int32(4 * h)), axis=1,
                            keepdims=True))
        P = jnp.where(eq, L, P)
        I = jnp.where(eq, LI, I)
        L = jnp.where(eq, jnp.float32(jnp.inf), L)
    idx_ref[...] = jnp.concatenate(cols, axis=1)
    x = _dot(feat_ref[...], fc1w_ref[...]) + fc1b_ref[...]
    e = _dot(xi, d1w_ref[...])
    x_ref[...] = jnp.concatenate([x, e], axis=1)
    qe_ref[...] = jnp.concatenate([_dot(x, wq_ref[...]), e], axis=1)


def _knn_call(xyzp, xTe, xTo, feat, fc1w, fc1b, wq, d1w):
    N = xyzp.shape[0]
    grid = (N // PB,)
    return pl.pallas_call(
        _knn_body,
        grid=grid,
        in_specs=[
            pl.BlockSpec((PB, 8), lambda i: (i, 0)),
            pl.BlockSpec((8, N // 2), lambda i: (0, 0)),
            pl.BlockSpec((8, N // 2), lambda i: (0, 0)),
            pl.BlockSpec((PB, D), lambda i: (i, 0)),
            pl.BlockSpec((D, D), lambda i: (0, 0)),
            pl.BlockSpec((1, D), lambda i: (0, 0)),
            pl.BlockSpec((D, D), lambda i: (0, 0)),
            pl.BlockSpec((8, D), lambda i: (0, 0)),
        ],
        out_specs=[
            pl.BlockSpec((PB, K), lambda i: (i, 0)),
            pl.BlockSpec((PB, 128), lambda i: (i, 0)),
            pl.BlockSpec((PB, 128), lambda i: (i, 0)),
        ],
        out_shape=[
            jax.ShapeDtypeStruct((N, K), jnp.int32),
            jax.ShapeDtypeStruct((N, 128), jnp.float32),
            jax.ShapeDtypeStruct((N, 128), jnp.float32),
        ],
    )(xyzp, xTe, xTo, feat, fc1w, fc1b, wq, d1w)


def _sc_gather(xt, idx2d):
    Ri = idx2d.shape[0]
    W = xt.shape[1]
    M = Ri * 128
    info = plsc.get_sparse_core_info()
    NW = info.num_cores * info.num_subcores
    rows_per_w = Ri // NW
    mesh = plsc.VectorSubcoreMesh(core_axis_name="c", subcore_axis_name="s")

    @functools.partial(
        pl.kernel,
        mesh=mesh,
        out_type=jax.ShapeDtypeStruct((M, W), jnp.float32),
        scratch_types=[
            pltpu.VMEM((128,), jnp.int32),
            pltpu.VMEM((128, W), jnp.float32),
            pltpu.VMEM((128, W), jnp.float32),
            pltpu.SemaphoreType.DMA,
            pltpu.SemaphoreType.DMA,
            pltpu.SemaphoreType.DMA,
        ],
    )
    def gather_k(xt_hbm, idx_hbm, gx_out, idx_v, rx0, rx1, gsem, osem0, osem1):
        wid = lax.axis_index("s") * info.num_cores + lax.axis_index("c")
        base = wid * rows_per_w
        rx = (rx0, rx1)
        osem = (osem0, osem1)
        pending = [None, None]
        for i in range(rows_per_w):
            p = i % 2
            if pending[p] is not None:
                pending[p].wait()
            r = base + i
            pltpu.sync_copy(idx_hbm.at[r], idx_v)
            pltpu.async_copy(xt_hbm.at[idx_v], rx[p], gsem).wait()
            cp = pltpu.async_copy(rx[p], gx_out.at[pl.ds(r * 128, 128)],
                                  osem[p])
            pending[p] = cp
        pending[0].wait()
        pending[1].wait()

    return gather_k(xt, idx2d)


def _rep16(a, pp):
    c = a.shape[-1]
    return jnp.broadcast_to(a[:, None, :], (pp, K, c)).reshape(pp * K, c)


def _dense_body(qe_ref, gx_ref, feat_ref,
                wk_ref, wv_ref, d1b_ref, d2w_ref, d2b_ref,
                g1w_ref, g1b_ref, g2w_ref, g2b_ref, fc2w_ref, fc2b_ref,
                attn_ref, res_ref):
    xj = gx_ref[:, :D]
    ej = gx_ref[:, D:]
    kj = _dot(xj, wk_ref[...])
    vj = _dot(xj, wv_ref[...])
    ei = _rep16(qe_ref[:, D:], PP)
    pos = jnp.maximum(ei - ej + d1b_ref[...], 0.0)
    pos = _dot(pos, d2w_ref[...]) + d2b_ref[...]
    g = _rep16(qe_ref[:, :D], PP) - kj + pos
    a = jnp.maximum(_dot(g, g1w_ref[...]) + g1b_ref[...], 0.0)
    a = _dot(a, g2w_ref[...]) + g2b_ref[...]
    a3 = a.reshape(PP, K, D) * jnp.float32(0.125)
    m = jnp.max(a3, axis=1, keepdims=True)
    e = jnp.exp(a3 - m)
    s = jnp.sum(e, axis=1, keepdims=True)
    p3 = e / s
    attn_ref[...] = p3
    w = p3 * (vj + pos).reshape(PP, K, D)
    r = jnp.sum(w, axis=1)
    res_ref[...] = _dot(r, fc2w_ref[...]) + fc2b_ref[...] + feat_ref[...]


def _dense_call(qe, gx, feat2, p):
    N = qe.shape[0]
    grid = (N // PP,)
    wfull = lambda shape: pl.BlockSpec(shape, lambda i: (0, 0))
    return pl.pallas_call(
        _dense_body,
        grid=grid,
        in_specs=[
            pl.BlockSpec((PP, 128), lambda i: (i, 0)),
            pl.BlockSpec((PP * K, 128), lambda i: (i, 0)),
            pl.BlockSpec((PP, D), lambda i: (i, 0)),
            wfull((D, D)), wfull((D, D)),
            wfull((1, D)), wfull((D, D)), wfull((1, D)),
            wfull((D, D)), wfull((1, D)), wfull((D, D)), wfull((1, D)),
            wfull((D, D)), wfull((1, D)),
        ],
        out_specs=[
            pl.BlockSpec((PP, K, D), lambda i: (i, 0, 0)),
            pl.BlockSpec((PP, D), lambda i: (i, 0)),
        ],
        out_shape=[
            jax.ShapeDtypeStruct((N, K, D), jnp.float32),
            jax.ShapeDtypeStruct((N, D), jnp.float32),
        ],
    )(qe, gx, feat2,
      p['wk'], p['wv'],
      p['delta1_b'], p['delta2_w'], p['delta2_b'],
      p['gamma1_w'], p['gamma1_b'], p['gamma2_w'], p['gamma2_b'],
      p['fc2_w'], p['fc2_b'])


def kernel(xyz, features, params):
    p = params
    B, N, _ = xyz.shape
    xyzp = jnp.pad(xyz, ((0, 0), (0, 0), (0, 5)))
    xyzT = jnp.swapaxes(xyzp, 1, 2)
    fc1b = p['fc1_b'].reshape(1, D)
    d1w = jnp.pad(p['delta1_w'], ((0, 5), (0, 0)))
    pr = {
        'wk': p['wk'], 'wv': p['wv'],
        'delta1_b': p['delta1_b'].reshape(1, D),
        'delta2_w': p['delta2_w'], 'delta2_b': p['delta2_b'].reshape(1, D),
        'gamma1_w': p['gamma1_w'], 'gamma1_b': p['gamma1_b'].reshape(1, D),
        'gamma2_w': p['gamma2_w'], 'gamma2_b': p['gamma2_b'].reshape(1, D),
        'fc2_w': p['fc2_w'], 'fc2_b': p['fc2_b'].reshape(1, D),
    }

    xTe = xyzT[:, :, 0::2]
    xTo = xyzT[:, :, 1::2]
    knn = [_knn_call(xyzp[b], xTe[b], xTo[b], features[b], p['fc1_w'], fc1b,
                     p['wq'], d1w) for b in range(B)]
    gx = [_sc_gather(knn[b][1], knn[b][0].reshape(N * K // 128, 128))
          for b in range(B)]
    outs = [_dense_call(knn[b][2], gx[b], features[b], pr) for b in range(B)]

    res = jnp.stack([o[1] for o in outs])
    attn = jnp.stack([o[0] for o in outs])
    return res, attn


# speedup vs baseline: 1.3831x; 1.3831x over previous
import functools

import jax
import jax.numpy as jnp
from jax import lax
from jax.experimental import pallas as pl
from jax.experimental.pallas import tpu as pltpu
from jax.experimental.pallas import tpu_sc as plsc

D = 64
K = 16
PB = 256
PP = 256

_DEF = jax.lax.Precision.DEFAULT


def _dot(a, b):
    return jax.lax.dot_general(
        a, b, (((1,), (0,)), ((), ())),
        precision=_DEF, preferred_element_type=jnp.float32)


def _knn_body(xyzp_ref, xyzT_ref, feat_ref, fc1w_ref, fc1b_ref,
              wq_ref, d1w_ref, idx_ref, x_ref, qe_ref):
    n = xyzT_ref.shape[1]
    xi = xyzp_ref[...]
    xT = xyzT_ref[...]
    cross = _dot(xi, xT)
    rsum_j = jnp.sum(xT * xT, axis=0, keepdims=True)
    rsum_i = jnp.sum(xi * xi, axis=1, keepdims=True)
    d = rsum_i + rsum_j - 2.0 * cross
    iota = jax.lax.broadcasted_iota(jnp.int32, (PB, n), 1)
    cols = []
    for _ in range(K):
        idx = jnp.argmin(d, axis=1)[:, None]
        cols.append(idx)
        d = jnp.where(iota == idx, jnp.float32(jnp.inf), d)
    idx_ref[...] = jnp.concatenate(cols, axis=1)
    x = _dot(feat_ref[...], fc1w_ref[...]) + fc1b_ref[...]
    e = _dot(xi, d1w_ref[...])
    x_ref[...] = jnp.concatenate([x, e], axis=1)
    qe_ref[...] = jnp.concatenate([_dot(x, wq_ref[...]), e], axis=1)


def _knn_call(xyzp, xyzT, feat, fc1w, fc1b, wq, d1w):
    N = xyzp.shape[0]
    grid = (N // PB,)
    return pl.pallas_call(
        _knn_body,
        grid=grid,
        in_specs=[
            pl.BlockSpec((PB, 8), lambda i: (i, 0)),
            pl.BlockSpec((8, N), lambda i: (0, 0)),
            pl.BlockSpec((PB, D), lambda i: (i, 0)),
            pl.BlockSpec((D, D), lambda i: (0, 0)),
            pl.BlockSpec((1, D), lambda i: (0, 0)),
            pl.BlockSpec((D, D), lambda i: (0, 0)),
            pl.BlockSpec((8, D), lambda i: (0, 0)),
        ],
        out_specs=[
            pl.BlockSpec((PB, K), lambda i: (i, 0)),
            pl.BlockSpec((PB, 128), lambda i: (i, 0)),
            pl.BlockSpec((PB, 128), lambda i: (i, 0)),
        ],
        out_shape=[
            jax.ShapeDtypeStruct((N, K), jnp.int32),
            jax.ShapeDtypeStruct((N, 128), jnp.float32),
            jax.ShapeDtypeStruct((N, 128), jnp.float32),
        ],
    )(xyzp, xyzT, feat, fc1w, fc1b, wq, d1w)


def _sc_gather(xt, idx2d):
    Ri = idx2d.shape[0]
    W = xt.shape[1]
    M = Ri * 128
    info = plsc.get_sparse_core_info()
    NW = info.num_cores * info.num_subcores
    rows_per_w = Ri // NW
    mesh = plsc.VectorSubcoreMesh(core_axis_name="c", subcore_axis_name="s")

    @functools.partial(
        pl.kernel,
        mesh=mesh,
        out_type=jax.ShapeDtypeStruct((M, W), jnp.float32),
        scratch_types=[
            pltpu.VMEM((128,), jnp.int32),
            pltpu.VMEM((128, W), jnp.float32),
            pltpu.VMEM((128, W), jnp.float32),
            pltpu.SemaphoreType.DMA,
            pltpu.SemaphoreType.DMA,
            pltpu.SemaphoreType.DMA,
        ],
    )
    def gather_k(xt_hbm, idx_hbm, gx_out, idx_v, rx0, rx1, gsem, osem0, osem1):
        wid = lax.axis_index("s") * info.num_cores + lax.axis_index("c")
        base = wid * rows_per_w
        rx = (rx0, rx1)
        osem = (osem0, osem1)
        pending = [None, None]
        for i in range(rows_per_w):
            p = i % 2
            if pending[p] is not None:
                pending[p].wait()
            r = base + i
            pltpu.sync_copy(idx_hbm.at[r], idx_v)
            pltpu.async_copy(xt_hbm.at[idx_v], rx[p], gsem).wait()
            cp = pltpu.async_copy(rx[p], gx_out.at[pl.ds(r * 128, 128)],
                                  osem[p])
            pending[p] = cp
        pending[0].wait()
        pending[1].wait()

    return gather_k(xt, idx2d)


def _rep16(a, pp):
    c = a.shape[-1]
    return jnp.broadcast_to(a[:, None, :], (pp, K, c)).reshape(pp * K, c)


def _dense_body(qe_ref, gx_ref, feat_ref,
                wk_ref, wv_ref, d1b_ref, d2w_ref, d2b_ref,
                g1w_ref, g1b_ref, g2w_ref, g2b_ref, fc2w_ref, fc2b_ref,
                attn_ref, res_ref):
    xj = gx_ref[:, :D]
    ej = gx_ref[:, D:]
    kj = _dot(xj, wk_ref[...])
    vj = _dot(xj, wv_ref[...])
    ei = _rep16(qe_ref[:, D:], PP)
    pos = jnp.maximum(ei - ej + d1b_ref[...], 0.0)
    pos = _dot(pos, d2w_ref[...]) + d2b_ref[...]
    g = _rep16(qe_ref[:, :D], PP) - kj + pos
    a = jnp.maximum(_dot(g, g1w_ref[...]) + g1b_ref[...], 0.0)
    a = _dot(a, g2w_ref[...]) + g2b_ref[...]
    a3 = a.reshape(PP, K, D) * jnp.float32(0.125)
    m = jnp.max(a3, axis=1, keepdims=True)
    e = jnp.exp(a3 - m)
    s = jnp.sum(e, axis=1, keepdims=True)
    p3 = e / s
    attn_ref[...] = p3
    w = p3 * (vj + pos).reshape(PP, K, D)
    r = jnp.sum(w, axis=1)
    res_ref[...] = _dot(r, fc2w_ref[...]) + fc2b_ref[...] + feat_ref[...]


def _dense_call(qe, gx, feat2, p):
    N = qe.shape[0]
    grid = (N // PP,)
    wfull = lambda shape: pl.BlockSpec(shape, lambda i: (0, 0))
    return pl.pallas_call(
        _dense_body,
        grid=grid,
        in_specs=[
            pl.BlockSpec((PP, 128), lambda i: (i, 0)),
            pl.BlockSpec((PP * K, 128), lambda i: (i, 0)),
            pl.BlockSpec((PP, D), lambda i: (i, 0)),
            wfull((D, D)), wfull((D, D)),
            wfull((1, D)), wfull((D, D)), wfull((1, D)),
            wfull((D, D)), wfull((1, D)), wfull((D, D)), wfull((1, D)),
            wfull((D, D)), wfull((1, D)),
        ],
        out_specs=[
            pl.BlockSpec((PP, K, D), lambda i: (i, 0, 0)),
            pl.BlockSpec((PP, D), lambda i: (i, 0)),
        ],
        out_shape=[
            jax.ShapeDtypeStruct((N, K, D), jnp.float32),
            jax.ShapeDtypeStruct((N, D), jnp.float32),
        ],
    )(qe, gx, feat2,
      p['wk'], p['wv'],
      p['delta1_b'], p['delta2_w'], p['delta2_b'],
      p['gamma1_w'], p['gamma1_b'], p['gamma2_w'], p['gamma2_b'],
      p['fc2_w'], p['fc2_b'])


def kernel(xyz, features, params):
    p = params
    B, N, _ = xyz.shape
    xyzp = jnp.pad(xyz, ((0, 0), (0, 0), (0, 5)))
    xyzT = jnp.swapaxes(xyzp, 1, 2)
    fc1b = p['fc1_b'].reshape(1, D)
    d1w = jnp.pad(p['delta1_w'], ((0, 5), (0, 0)))
    pr = {
        'wk': p['wk'], 'wv': p['wv'],
        'delta1_b': p['delta1_b'].reshape(1, D),
        'delta2_w': p['delta2_w'], 'delta2_b': p['delta2_b'].reshape(1, D),
        'gamma1_w': p['gamma1_w'], 'gamma1_b': p['gamma1_b'].reshape(1, D),
        'gamma2_w': p['gamma2_w'], 'gamma2_b': p['gamma2_b'].reshape(1, D),
        'fc2_w': p['fc2_w'], 'fc2_b': p['fc2_b'].reshape(1, D),
    }

    knn = [_knn_call(xyzp[b], xyzT[b], features[b], p['fc1_w'], fc1b,
                     p['wq'], d1w) for b in range(B)]
    gx = [_sc_gather(knn[b][1], knn[b][0].reshape(N * K // 128, 128))
          for b in range(B)]
    outs = [_dense_call(knn[b][2], gx[b], features[b], pr) for b in range(B)]

    res = jnp.stack([o[1] for o in outs])
    attn = jnp.stack([o[0] for o in outs])
    return res, attn
